# baseline (speedup 1.0000x reference)
import jax
import jax.numpy as jnp
from jax.experimental import pallas as pl


def _add_kernel(x_ref, emb_ref, out_ref):
    out_ref[...] = x_ref[...] + emb_ref[...][None]


def kernel(x, embedding):
    b, c, d = x.shape
    bc = 256
    nc = c // bc

    grid = (nc,)
    return pl.pallas_call(
        _add_kernel,
        grid=grid,
        in_specs=[
            pl.BlockSpec((b, bc, d), lambda ci: (0, ci, 0)),
            pl.BlockSpec((bc, d), lambda ci: (ci, 0)),
        ],
        out_specs=pl.BlockSpec((b, bc, d), lambda ci: (0, ci, 0)),
        out_shape=jax.ShapeDtypeStruct((b, c, d), x.dtype),
    )(x, embedding)
